# baseline (speedup 1.0000x reference)
import functools

import jax
import jax.numpy as jnp
from jax import lax
from jax.experimental import pallas as pl
from jax.experimental.pallas import tpu as pltpu
from jax.experimental.pallas import tpu_sc as plsc

NC = 2
NS = 16
NW = NC * NS
ROWS = 65536
COLS = 100
MCOLS = 10
K = ROWS // MCOLS + 1
KPW = 208
KPAD = NW * KPW // MCOLS * MCOLS
KCOLS = NW * KPW
KW = 384
NG = KPW // 16


def _sc_partials(xt, b):
    mesh = plsc.VectorSubcoreMesh(core_axis_name="c", subcore_axis_name="s")

    @functools.partial(
        pl.kernel,
        out_type=jax.ShapeDtypeStruct((NW, 16), jnp.float32),
        mesh=mesh,
        scratch_types=[
            pltpu.VMEM((COLS, KW), jnp.float32),
            pltpu.VMEM((COLS, KW), jnp.float32),
            pltpu.VMEM((16,), jnp.float32),
        ],
    )
    def k(xt_hbm, b_hbm, out_hbm, a_v, b_v, res_v):
        wid = lax.axis_index("s") * NC + lax.axis_index("c")
        kw0 = wid * KPW
        kend = jnp.minimum(kw0 + KPW, K)
        col0 = kw0 // 128 * 128
        pltpu.sync_copy(xt_hbm.at[:, pl.ds(col0, KW)], a_v)
        pltpu.sync_copy(b_hbm.at[:, pl.ds(col0, KW)], b_v)

        off0 = kw0 - col0
        iota = lax.iota(jnp.int32, 16)
        masks = [kw0 + 16 * g + iota < kend for g in range(NG)]

        def crow(c, acc):
            for g in range(NG):
                va = a_v[c, pl.ds(off0 + 16 * g, 16)]
                vb = b_v[c, pl.ds(off0 + 16 * g, 16)]
                acc = acc + jnp.where(masks[g], jnp.abs(va - vb), 0.0)
            return acc

        acc = lax.fori_loop(0, COLS, crow, jnp.zeros((16,), jnp.float32))
        res_v[...] = acc
        pltpu.sync_copy(res_v, out_hbm.at[wid])

    return k(xt, b)


def kernel(x):
    xt = x.T
    patch = xt[6 * MCOLS:COLS, K - 1].reshape(4, MCOLS).T
    x10 = jnp.concatenate(
        [xt[:MCOLS], patch,
         jnp.zeros((MCOLS, KCOLS * MCOLS - ROWS - 4), jnp.float32)], axis=1)
    b = (x10.reshape(MCOLS, KCOLS, MCOLS)
         .transpose(2, 0, 1)
         .reshape(COLS, KCOLS))
    partials = _sc_partials(xt, b)
    return jnp.sum(partials)
